# Initial kernel scaffold; baseline (speedup 1.0000x reference)
import jax
import jax.numpy as jnp
from jax.experimental import pallas as pl


def kernel(x):
    raise NotImplementedError("write your pallas kernel here")


# speedup vs baseline: 63.1634x; 63.1634x over previous
import functools

import jax
import jax.numpy as jnp
from jax.experimental import pallas as pl
from jax.experimental.pallas import tpu as pltpu

_KEEP_FRAC = 0.1


def _select_kernel(k, x_ref, o_ref):
    x = x_ref[...]
    keys = jax.lax.bitcast_convert_type(x, jnp.int32) & jnp.int32(0x7FFFFFFF)

    def body(i, lo):
        cand = lo | jax.lax.shift_left(jnp.int32(1), jnp.int32(30) - i)
        cnt = jnp.sum((keys >= cand).astype(jnp.int32))
        return jnp.where(cnt >= k, cand, lo)

    thr = jax.lax.fori_loop(0, 31, body, jnp.int32(0))
    o_ref[...] = jnp.where(keys >= thr, x, jnp.float32(0.0))


def kernel(x):
    B, C, L = x.shape
    n = C * L
    k = max(1, int(round(_KEEP_FRAC * n)))
    return pl.pallas_call(
        functools.partial(_select_kernel, k),
        grid=(B,),
        in_specs=[pl.BlockSpec((1, C, L), lambda b: (b, 0, 0))],
        out_specs=pl.BlockSpec((1, C, L), lambda b: (b, 0, 0)),
        out_shape=jax.ShapeDtypeStruct(x.shape, x.dtype),
    )(x)
